# baseline (speedup 1.0000x reference)
import functools
import jax
import jax.numpy as jnp
from jax import lax
from jax.experimental import pallas as pl
from jax.experimental.pallas import tpu as pltpu
from jax.experimental.pallas import tpu_sc as plsc

N = 10000; T = 16; E = 160000; B = 8
D_ERA = 16; D_BC = 32; D_BD = 16; D_RC = 16; D_RD = 8
H = 128; LH = 256; K = 3

NC = 2
NS = 16
EC = E // NS
ECP = 10112
NG = ECP // 32
ECA = ECP + 128
HC = 80
NP = 10240
NPT = NP // NS


def _sc1_body(xA, xB, s1s, s1d, m1, srcp, dstp,
              p1parts,
              srcc, dstc, s1sv, s1dv, m1v,
              idxS, idxD, rows, scaled,
              zb, gsems, ssems,
              p1acc):
    c = lax.axis_index("c")
    s = lax.axis_index("s")
    z16 = jnp.zeros((16,), jnp.float32)
    iota = lax.iota(jnp.int32, 16)
    D = 4

    pltpu.sync_copy(srcp.at[s], srcc)
    pltpu.sync_copy(dstp.at[s], dstc)

    def zero_zb(i, _):
        for k in range(HC // 16):
            zb[i, pl.ds(k * 16, 16)] = z16
        return _
    lax.fori_loop(0, 80, zero_zb, None)

    def gissue(g, toff, b):
        idxS[b, pl.ds(0, 16)] = srcc[pl.ds(g * 32, 16)] + toff
        idxS[b, pl.ds(16, 16)] = srcc[pl.ds(g * 32 + 16, 16)] + toff
        @pl.when(c == 0)
        def _():
            pltpu.async_copy(xA.at[idxS.at[b]], rows.at[b], gsems.at[b])
        @pl.when(c == 1)
        def _():
            pltpu.async_copy(xB.at[idxS.at[b]], rows.at[b], gsems.at[b])

    def gwait(b):
        @pl.when(c == 0)
        def _():
            pltpu.make_async_copy(xA.at[idxS.at[b]], rows.at[b],
                                  gsems.at[b]).wait()
        @pl.when(c == 1)
        def _():
            pltpu.make_async_copy(xB.at[idxS.at[b]], rows.at[b],
                                  gsems.at[b]).wait()

    def swait(b):
        pltpu.make_async_copy(scaled.at[b], p1acc.at[idxD.at[b]],
                              ssems.at[b]).wait()

    def t_body(t, _):
        pltpu.sync_copy(s1s.at[t], s1sv)
        pltpu.sync_copy(s1d.at[t], s1dv)
        pltpu.sync_copy(m1.at[t], m1v)
        m1t = m1v[...]
        toff = t * N

        for j in range(8):
            dst_off = pl.multiple_of(s * NPT + j * 80, 8)
            pltpu.sync_copy(zb, p1acc.at[pl.ds(dst_off, 80), :])
        plsc.subcore_barrier()

        for b in range(D):
            gissue(b, toff, b)

        def quad_body(q, _):
            for b in range(D):
                g = q * D + b
                gwait(b)
                @pl.when(q > 0)
                def _():
                    swait(b)
                for h in range(2):
                    off = g * 32 + h * 16
                    src16 = srcc[pl.ds(off, 16)]
                    dst16 = dstc[pl.ds(off, 16)]
                    s_s = plsc.load_gather(s1sv, [src16])
                    s_d = plsc.load_gather(s1dv, [dst16])
                    e = s_s + s_d
                    e = jnp.where(e >= 0.0, e, e * 0.2)
                    ex = jnp.exp(e - m1t)
                    ex = jnp.where((off + iota) < EC, ex, 0.0)
                    for j in range(16):
                        exj = ex[j]
                        for k in range(HC // 16):
                            sl = pl.ds(k * 16, 16)
                            scaled[b, h * 16 + j, sl] = rows[b, h * 16 + j, sl] * exj
                    idxD[b, pl.ds(h * 16, 16)] = dst16
                pltpu.async_copy(scaled.at[b], p1acc.at[idxD.at[b]],
                                 ssems.at[b], add=True)
                gissue(g + D, toff, b)
            return _
        lax.fori_loop(0, NG // D, quad_body, None)
        for b in range(D):
            gwait(b)
            swait(b)
        plsc.subcore_barrier()

        src_off = pl.multiple_of(s * NPT, 8)
        pltpu.sync_copy(p1acc.at[pl.ds(src_off, NPT), :],
                        p1parts.at[c, t, pl.ds(src_off, NPT), :])
        return _
    lax.fori_loop(0, T, t_body, None)


def _sc1_layer1(xA, xB, s1s, s1d, m1, srcp, dstp):
    mesh = plsc.VectorSubcoreMesh(core_axis_name="c", subcore_axis_name="s")
    f = pl.kernel(
        _sc1_body,
        out_type=jax.ShapeDtypeStruct((NC, T, NP, HC), jnp.float32),
        mesh=mesh,
        compiler_params=pltpu.CompilerParams(needs_layout_passes=False,
                                             use_tc_tiling_on_sc=False),
        scratch_types=[
            pltpu.VMEM((ECA,), jnp.int32),
            pltpu.VMEM((ECA,), jnp.int32),
            pltpu.VMEM((N,), jnp.float32),
            pltpu.VMEM((N,), jnp.float32),
            pltpu.VMEM((16,), jnp.float32),
            pltpu.VMEM((4, 32), jnp.int32),
            pltpu.VMEM((4, 32), jnp.int32),
            pltpu.VMEM((4, 32, HC), jnp.float32),
            pltpu.VMEM((4, 32, HC), jnp.float32),
            pltpu.VMEM((80, HC), jnp.float32),
            pltpu.SemaphoreType.DMA((4,)),
            pltpu.SemaphoreType.DMA((4,)),
            pltpu.VMEM_SHARED((NP, HC), jnp.float32),
        ],
    )
    return f(xA, xB, s1s, s1d, m1, srcp, dstp)


NGC = 313


def _sc2_body(h1flat, s2s, s2d, m2, srcp, dstp, bidx,
              out2parts, den2parts,
              srcc, dstc, s2sv, s2dv, m2v, bidxv, den2v,
              idxS, slotb, rows, scaled, zb2, sem,
              out2acc):
    c = lax.axis_index("c")
    s = lax.axis_index("s")
    z16 = jnp.zeros((16,), jnp.float32)
    iota = lax.iota(jnp.int32, 16)

    pltpu.sync_copy(srcp.at[s], srcc)
    pltpu.sync_copy(dstp.at[s], dstc)
    pltpu.sync_copy(bidx, bidxv)
    for k in range(16):
        for q in range(8):
            zb2[k, pl.ds(q * 16, 16)] = z16
    bvec = bidxv[...]
    base = c * (EC // 2)
    limit = base + (EC // 2)

    def t_body(t, _):
        pltpu.sync_copy(s2s.at[t], s2sv)
        pltpu.sync_copy(s2d.at[t], s2dv)
        pltpu.sync_copy(m2.at[t], m2v)
        m2t = m2v[...]
        toff = t * N

        @pl.when(s == 0)
        def _():
            pltpu.sync_copy(zb2, out2acc)
        plsc.subcore_barrier()

        def g_body(g, accs):
            off = base + g * 16
            dst16 = dstc[pl.ds(off, 16)]
            valid = (off + iota) < limit
            hit = valid
            m = jnp.zeros((16,), jnp.bool_)
            for j in range(8):
                m = m | (dst16 == bvec[j])
            hit = hit & m
            anyhit = jnp.max(jnp.where(hit, 1, 0), axis=0)

            def do_group(accs):
                src16 = srcc[pl.ds(off, 16)]
                slot = jnp.full((16,), 8, jnp.int32)
                for j in range(7, -1, -1):
                    slot = jnp.where(dst16 == bvec[j], j, slot)
                s_s = plsc.load_gather(s2sv, [src16])
                s_d = plsc.load_gather(s2dv, [slot])
                e = s_s + s_d
                e = jnp.where(e >= 0.0, e, e * 0.2)
                ex = jnp.exp(e - m2t)
                ex = jnp.where(hit, ex, 0.0)
                naccs = tuple(accs[j] + jnp.where(slot == j, ex, 0.0)
                              for j in range(8))
                idxS[...] = src16 + toff
                slotb[...] = slot
                pltpu.async_copy(h1flat.at[idxS], rows, sem).wait()
                for j in range(16):
                    exj = ex[j]
                    for k in range(8):
                        sl = pl.ds(k * 16, 16)
                        scaled[j, sl] = rows[j, sl] * exj
                pltpu.sync_copy(scaled, out2acc.at[slotb], add=True)
                return naccs

            def skip(accs):
                return accs
            return lax.cond(anyhit > 0, do_group, skip, accs)

        accs = tuple(z16 for _ in range(8))
        accs = lax.fori_loop(0, NGC, g_body, accs)
        plsc.subcore_barrier()

        for j in range(8):
            den2v[j, pl.ds(0, 16)] = accs[j]
        pltpu.sync_copy(den2v, den2parts.at[c, s, t])

        @pl.when(s == 0)
        def _():
            pltpu.sync_copy(out2acc, out2parts.at[c, t])
        return _
    lax.fori_loop(0, T, t_body, None)


def _sc2_layer2(h1flat, s2s, s2d, m2, srcp, dstp, bidx):
    mesh = plsc.VectorSubcoreMesh(core_axis_name="c", subcore_axis_name="s")
    f = pl.kernel(
        _sc2_body,
        out_type=(
            jax.ShapeDtypeStruct((NC, T, 16, H), jnp.float32),
            jax.ShapeDtypeStruct((NC, NS, T, 8, 16), jnp.float32),
        ),
        mesh=mesh,
        compiler_params=pltpu.CompilerParams(needs_layout_passes=False,
                                             use_tc_tiling_on_sc=False),
        scratch_types=[
            pltpu.VMEM((ECA,), jnp.int32),
            pltpu.VMEM((ECA,), jnp.int32),
            pltpu.VMEM((N,), jnp.float32),
            pltpu.VMEM((16,), jnp.float32),
            pltpu.VMEM((16,), jnp.float32),
            pltpu.VMEM((16,), jnp.int32),
            pltpu.VMEM((8, 16), jnp.float32),
            pltpu.VMEM((16,), jnp.int32),
            pltpu.VMEM((16,), jnp.int32),
            pltpu.VMEM((16, H), jnp.float32),
            pltpu.VMEM((16, H), jnp.float32),
            pltpu.VMEM((16, H), jnp.float32),
            pltpu.SemaphoreType.DMA,
            pltpu.VMEM_SHARED((16, H), jnp.float32),
        ],
    )
    return f(h1flat, s2s, s2d, m2, srcp, dstp, bidx)


NB = 1000


def _proj_body(era5_ref, bc_ref, bd_ref, wera_ref, wbc_ref, wbd_ref, bias_ref,
               v1s_ref, v1d_ref, xA_ref, xB_ref, s1s_ref, s1d_ref):
    x = (jnp.dot(era5_ref[0], wera_ref[...], preferred_element_type=jnp.float32)
         + jnp.dot(bc_ref[...], wbc_ref[...], preferred_element_type=jnp.float32)
         + jnp.dot(bd_ref[...], wbd_ref[...], preferred_element_type=jnp.float32)
         + bias_ref[...])
    x = jax.nn.relu(x)
    z16c = jnp.zeros((NB, 16), jnp.float32)
    xA_ref[0] = jnp.concatenate([x[:, :64], z16c], axis=1)
    xB_ref[0] = jnp.concatenate([x[:, 64:], jnp.ones((NB, 1), jnp.float32),
                                 z16c[:, :15]], axis=1)
    s1s_ref[0] = jnp.dot(x, v1s_ref[...], preferred_element_type=jnp.float32)
    s1d_ref[0] = jnp.dot(x, v1d_ref[...], preferred_element_type=jnp.float32)


def _proj(era5t, bc, bd, wera, wbc, wbd, bias, v1s, v1d):
    return pl.pallas_call(
        _proj_body,
        grid=(T, N // NB),
        in_specs=[
            pl.BlockSpec((1, NB, D_ERA), lambda t, nb: (t, nb, 0)),
            pl.BlockSpec((NB, D_BC), lambda t, nb: (nb, 0)),
            pl.BlockSpec((NB, D_BD), lambda t, nb: (nb, 0)),
            pl.BlockSpec((D_ERA, H), lambda t, nb: (0, 0)),
            pl.BlockSpec((D_BC, H), lambda t, nb: (0, 0)),
            pl.BlockSpec((D_BD, H), lambda t, nb: (0, 0)),
            pl.BlockSpec((1, H), lambda t, nb: (0, 0)),
            pl.BlockSpec((H, 1), lambda t, nb: (0, 0)),
            pl.BlockSpec((H, 1), lambda t, nb: (0, 0)),
        ],
        out_specs=[
            pl.BlockSpec((1, NB, HC), lambda t, nb: (t, nb, 0)),
            pl.BlockSpec((1, NB, HC), lambda t, nb: (t, nb, 0)),
            pl.BlockSpec((1, NB, 1), lambda t, nb: (t, nb, 0)),
            pl.BlockSpec((1, NB, 1), lambda t, nb: (t, nb, 0)),
        ],
        out_shape=[
            jax.ShapeDtypeStruct((T, N, HC), jnp.float32),
            jax.ShapeDtypeStruct((T, N, HC), jnp.float32),
            jax.ShapeDtypeStruct((T, N, 1), jnp.float32),
            jax.ShapeDtypeStruct((T, N, 1), jnp.float32),
        ],
    )(era5t, bc, bd, wera, wbc, wbd, bias, v1s, v1d)


def _h1_body(pa_ref, pb_ref, w_ref, b_ref, v2s_ref, h1_ref, s2s_ref):
    pa = pa_ref[0]
    pb = pb_ref[0]
    den = pb[:, 64:65] + 1e-16
    P1 = jnp.concatenate([pa[:, :64], pb[:, :64]], axis=1) / den
    h1 = jnp.dot(P1, w_ref[...], preferred_element_type=jnp.float32) + b_ref[...]
    h1 = jnp.where(h1 > 0, h1, jnp.exp(jnp.minimum(h1, 0.0)) - 1.0)
    h1_ref[0] = h1
    s2s_ref[0] = jnp.dot(h1, v2s_ref[...], preferred_element_type=jnp.float32)


def _h1k(pA, pB, w, b, v2s):
    return pl.pallas_call(
        _h1_body,
        grid=(T, N // NB),
        in_specs=[
            pl.BlockSpec((1, NB, HC), lambda t, nb: (t, nb, 0)),
            pl.BlockSpec((1, NB, HC), lambda t, nb: (t, nb, 0)),
            pl.BlockSpec((H, H), lambda t, nb: (0, 0)),
            pl.BlockSpec((1, H), lambda t, nb: (0, 0)),
            pl.BlockSpec((H, 1), lambda t, nb: (0, 0)),
        ],
        out_specs=[
            pl.BlockSpec((1, NB, H), lambda t, nb: (t, nb, 0)),
            pl.BlockSpec((1, NB, 1), lambda t, nb: (t, nb, 0)),
        ],
        out_shape=[
            jax.ShapeDtypeStruct((T, N, H), jnp.float32),
            jax.ShapeDtypeStruct((T, N, 1), jnp.float32),
        ],
    )(pA, pB, w, b, v2s)


def _lstm_head_body(series_ref, wih_ref, whh_ref, b_ref, hw_ref, hb_ref,
                    cast_ref, h_ref, c_ref):
    h = jnp.zeros((B, LH), jnp.float32)
    c = jnp.zeros((B, LH), jnp.float32)
    wih = wih_ref[...]
    whh = whh_ref[...]
    b = b_ref[...]
    hw = hw_ref[...]
    hb = hb_ref[...]
    for t in range(T):
        x_t = series_ref[t]
        z = jnp.dot(x_t, wih, preferred_element_type=jnp.float32) + \
            jnp.dot(h, whh, preferred_element_type=jnp.float32) + b
        i = jax.nn.sigmoid(z[:, 0 * LH:1 * LH])
        f = jax.nn.sigmoid(z[:, 1 * LH:2 * LH])
        g = jnp.tanh(z[:, 2 * LH:3 * LH])
        o = jax.nn.sigmoid(z[:, 3 * LH:4 * LH])
        c = f * c + i * g
        h = o * jnp.tanh(c)
        zc = jnp.dot(h, hw, preferred_element_type=jnp.float32) + hb
        m_ = zc[:, 0:K]
        b_ = jax.nn.softplus(zc[:, K:2 * K]) + 1e-5
        t_ = jax.nn.sigmoid(zc[:, 2 * K:3 * K])
        p_ = jax.nn.softmax(zc[:, 3 * K:4 * K], axis=-1)
        cast_ref[t] = jnp.concatenate([m_, b_, t_, p_], axis=-1)
    h_ref[...] = h
    c_ref[...] = c


def _lstm_head(series_tbh, W_ih, W_hh, b_lstm, head_W, head_b):
    return pl.pallas_call(
        _lstm_head_body,
        out_shape=(
            jax.ShapeDtypeStruct((T, B, 4 * K), jnp.float32),
            jax.ShapeDtypeStruct((B, LH), jnp.float32),
            jax.ShapeDtypeStruct((B, LH), jnp.float32),
        ),
    )(series_tbh, W_ih, W_hh, b_lstm, head_W, head_b)


def kernel(era5, basinContinuous, basinDiscrete, riverContinuous, riverDiscrete,
           bp_Wc, bp_bc, bp_Wd, bp_bd,
           g1_W, g1_as, g1_ad, g1_b, g2_W, g2_as, g2_ad, g2_b,
           rp_Wc, rp_bc, rp_Wd, rp_bd,
           W_ih, W_hh, b_lstm, head_W, head_b,
           edge_index, nodes):
    src, dst = edge_index[0], edge_index[1]

    era5t = jnp.swapaxes(era5, 0, 1)
    W_era = bp_Wc[:D_ERA]
    W_bc = bp_Wc[D_ERA:]
    bias = (bp_bc + bp_bd)[None]
    v1s = (g1_W @ g1_as)[:, None]
    v1d = (g1_W @ g1_ad)[:, None]
    xA3, xB3, s1s3, s1d3 = _proj(era5t, basinContinuous, basinDiscrete,
                                 W_era, W_bc, bp_Wd, bias, v1s, v1d)
    s1s = s1s3[..., 0]
    s1d = s1d3[..., 0]
    M1 = jnp.max(s1s, axis=1) + jnp.max(s1d, axis=1)
    m1bc = jnp.broadcast_to(M1[:, None], (T, 16))

    srcp = jnp.pad(src.reshape(NS, EC), ((0, 0), (0, ECA - EC)))
    dstp = jnp.pad(dst.reshape(NS, EC), ((0, 0), (0, ECA - EC)))
    p1parts = _sc1_layer1(xA3.reshape(T * N, HC), xB3.reshape(T * N, HC),
                          s1s, s1d, m1bc, srcp, dstp)

    v2s = (g2_W @ g2_as)[:, None]
    h1, s2s3 = _h1k(p1parts[0], p1parts[1], g1_W, g1_b[None], v2s)
    s2s = s2s3[..., 0]

    batchIndices = jnp.concatenate([jnp.zeros((1,), nodes.dtype), jnp.cumsum(nodes)[:-1]])

    v2d = g2_W @ g2_ad
    s2d_sel = h1[:, batchIndices, :] @ v2d
    M2 = jnp.max(s2s, axis=1) + jnp.max(s2d_sel, axis=1)

    s2d_pad = jnp.pad(s2d_sel, ((0, 0), (0, 8)))
    m2bc = jnp.broadcast_to(M2[:, None], (T, 16))
    bidx_pad = jnp.pad(batchIndices.astype(jnp.int32), (0, 8),
                       constant_values=-1)
    out2parts, den2parts = _sc2_layer2(h1.reshape(T * N, H), s2s, s2d_pad,
                                       m2bc, srcp, dstp, bidx_pad)
    agg = out2parts.sum(axis=0)[:, :8, :]
    denom2 = den2parts.sum(axis=(0, 1, 4))
    out2 = (agg / (denom2[..., None] + 1e-16)) @ g2_W + g2_b
    first = jnp.argmax(batchIndices[None, :] == batchIndices[:, None], axis=1)
    out2 = out2[:, first, :]

    rcat = jnp.concatenate([out2, jnp.broadcast_to(riverContinuous[None], (T, B, D_RC))], -1)
    series = jax.nn.relu(rcat @ rp_Wc + rp_bc + riverDiscrete @ rp_Wd + rp_bd)

    cast_t, h, c = _lstm_head(series, W_ih, W_hh, b_lstm, head_W, head_b)
    cast = jnp.swapaxes(cast_t, 0, 1)
    return cast, (h, c)


# speedup vs baseline: 1.0478x; 1.0478x over previous
import functools
import jax
import jax.numpy as jnp
from jax import lax
from jax.experimental import pallas as pl
from jax.experimental.pallas import tpu as pltpu
from jax.experimental.pallas import tpu_sc as plsc

N = 10000; T = 16; E = 160000; B = 8
D_ERA = 16; D_BC = 32; D_BD = 16; D_RC = 16; D_RD = 8
H = 128; LH = 256; K = 3

NC = 2
NS = 16
EC = E // NS
ECP = 10112
NG = ECP // 16
ECA = ECP + 128
HC = 80
NP = 10240
NPT = NP // NS


def _sc1_body(xA, xB, s1s, s1d, m1, srcp, dstp,
              p1parts,
              srcc, dstc, s1sv, s1dv, m1v,
              idxS, idxD, rows, scaled,
              zb, gsems, ssems,
              p1acc):
    c = lax.axis_index("c")
    s = lax.axis_index("s")
    z16 = jnp.zeros((16,), jnp.float32)
    iota = lax.iota(jnp.int32, 16)
    D = 8

    pltpu.sync_copy(srcp.at[s], srcc)
    pltpu.sync_copy(dstp.at[s], dstc)

    def zero_zb(i, _):
        for k in range(HC // 16):
            zb[i, pl.ds(k * 16, 16)] = z16
        return _
    lax.fori_loop(0, 80, zero_zb, None)

    def gissue(g, toff, b):
        idxS[b, pl.ds(0, 16)] = srcc[pl.ds(g * 16, 16)] + toff
        @pl.when(c == 0)
        def _():
            pltpu.async_copy(xA.at[idxS.at[b]], rows.at[b], gsems.at[b])
        @pl.when(c == 1)
        def _():
            pltpu.async_copy(xB.at[idxS.at[b]], rows.at[b], gsems.at[b])

    def gwait(b):
        @pl.when(c == 0)
        def _():
            pltpu.make_async_copy(xA.at[idxS.at[b]], rows.at[b],
                                  gsems.at[b]).wait()
        @pl.when(c == 1)
        def _():
            pltpu.make_async_copy(xB.at[idxS.at[b]], rows.at[b],
                                  gsems.at[b]).wait()

    def swait(b):
        pltpu.make_async_copy(scaled.at[b], p1acc.at[idxD.at[b]],
                              ssems.at[b]).wait()

    def t_body(t, _):
        pltpu.sync_copy(s1s.at[t], s1sv)
        pltpu.sync_copy(s1d.at[t], s1dv)
        pltpu.sync_copy(m1.at[t], m1v)
        m1t = m1v[...]
        toff = t * N

        for j in range(8):
            dst_off = pl.multiple_of(s * NPT + j * 80, 8)
            pltpu.sync_copy(zb, p1acc.at[pl.ds(dst_off, 80), :])
        plsc.subcore_barrier()

        for b in range(D):
            gissue(b, toff, b)

        def quad_body(q, _):
            for b in range(D):
                g = q * D + b
                gwait(b)
                @pl.when(q > 0)
                def _():
                    swait(b)
                off = g * 16
                src16 = srcc[pl.ds(off, 16)]
                dst16 = dstc[pl.ds(off, 16)]
                s_s = plsc.load_gather(s1sv, [src16])
                s_d = plsc.load_gather(s1dv, [dst16])
                e = s_s + s_d
                e = jnp.where(e >= 0.0, e, e * 0.2)
                ex = jnp.exp(e - m1t)
                ex = jnp.where((off + iota) < EC, ex, 0.0)
                for j in range(16):
                    exj = ex[j]
                    for k in range(HC // 16):
                        sl = pl.ds(k * 16, 16)
                        scaled[b, j, sl] = rows[b, j, sl] * exj
                idxD[b, pl.ds(0, 16)] = dst16
                pltpu.async_copy(scaled.at[b], p1acc.at[idxD.at[b]],
                                 ssems.at[b], add=True)
                gissue(g + D, toff, b)
            return _
        lax.fori_loop(0, NG // D, quad_body, None)
        for b in range(D):
            gwait(b)
            swait(b)
        plsc.subcore_barrier()

        src_off = pl.multiple_of(s * NPT, 8)
        pltpu.sync_copy(p1acc.at[pl.ds(src_off, NPT), :],
                        p1parts.at[c, t, pl.ds(src_off, NPT), :])
        return _
    lax.fori_loop(0, T, t_body, None)


def _sc1_layer1(xA, xB, s1s, s1d, m1, srcp, dstp):
    mesh = plsc.VectorSubcoreMesh(core_axis_name="c", subcore_axis_name="s")
    f = pl.kernel(
        _sc1_body,
        out_type=jax.ShapeDtypeStruct((NC, T, NP, HC), jnp.float32),
        mesh=mesh,
        compiler_params=pltpu.CompilerParams(needs_layout_passes=False,
                                             use_tc_tiling_on_sc=False),
        scratch_types=[
            pltpu.VMEM((ECA,), jnp.int32),
            pltpu.VMEM((ECA,), jnp.int32),
            pltpu.VMEM((N,), jnp.float32),
            pltpu.VMEM((N,), jnp.float32),
            pltpu.VMEM((16,), jnp.float32),
            pltpu.VMEM((8, 16), jnp.int32),
            pltpu.VMEM((8, 16), jnp.int32),
            pltpu.VMEM((8, 16, HC), jnp.float32),
            pltpu.VMEM((8, 16, HC), jnp.float32),
            pltpu.VMEM((80, HC), jnp.float32),
            pltpu.SemaphoreType.DMA((8,)),
            pltpu.SemaphoreType.DMA((8,)),
            pltpu.VMEM_SHARED((NP, HC), jnp.float32),
        ],
    )
    return f(xA, xB, s1s, s1d, m1, srcp, dstp)


NGC = 313


def _sc2_body(h1flat, s2s, s2d, m2, srcp, dstp, bidx,
              out2parts, den2parts,
              srcc, dstc, s2sv, s2dv, m2v, bidxv, den2v,
              idxS, slotb, rows, scaled, zb2, sem,
              out2acc):
    c = lax.axis_index("c")
    s = lax.axis_index("s")
    z16 = jnp.zeros((16,), jnp.float32)
    iota = lax.iota(jnp.int32, 16)

    pltpu.sync_copy(srcp.at[s], srcc)
    pltpu.sync_copy(dstp.at[s], dstc)
    pltpu.sync_copy(bidx, bidxv)
    for k in range(16):
        for q in range(8):
            zb2[k, pl.ds(q * 16, 16)] = z16
    bvec = bidxv[...]
    base = c * (EC // 2)
    limit = base + (EC // 2)

    def t_body(t, _):
        pltpu.sync_copy(s2s.at[t], s2sv)
        pltpu.sync_copy(s2d.at[t], s2dv)
        pltpu.sync_copy(m2.at[t], m2v)
        m2t = m2v[...]
        toff = t * N

        @pl.when(s == 0)
        def _():
            pltpu.sync_copy(zb2, out2acc)
        plsc.subcore_barrier()

        def g_body(g, accs):
            off = base + g * 16
            dst16 = dstc[pl.ds(off, 16)]
            valid = (off + iota) < limit
            hit = valid
            m = jnp.zeros((16,), jnp.bool_)
            for j in range(8):
                m = m | (dst16 == bvec[j])
            hit = hit & m
            anyhit = jnp.max(jnp.where(hit, 1, 0), axis=0)

            def do_group(accs):
                src16 = srcc[pl.ds(off, 16)]
                slot = jnp.full((16,), 8, jnp.int32)
                for j in range(7, -1, -1):
                    slot = jnp.where(dst16 == bvec[j], j, slot)
                s_s = plsc.load_gather(s2sv, [src16])
                s_d = plsc.load_gather(s2dv, [slot])
                e = s_s + s_d
                e = jnp.where(e >= 0.0, e, e * 0.2)
                ex = jnp.exp(e - m2t)
                ex = jnp.where(hit, ex, 0.0)
                naccs = tuple(accs[j] + jnp.where(slot == j, ex, 0.0)
                              for j in range(8))
                idxS[...] = src16 + toff
                slotb[...] = slot
                pltpu.async_copy(h1flat.at[idxS], rows, sem).wait()
                for j in range(16):
                    exj = ex[j]
                    for k in range(8):
                        sl = pl.ds(k * 16, 16)
                        scaled[j, sl] = rows[j, sl] * exj
                pltpu.sync_copy(scaled, out2acc.at[slotb], add=True)
                return naccs

            def skip(accs):
                return accs
            return lax.cond(anyhit > 0, do_group, skip, accs)

        accs = tuple(z16 for _ in range(8))
        accs = lax.fori_loop(0, NGC, g_body, accs)
        plsc.subcore_barrier()

        for j in range(8):
            den2v[j, pl.ds(0, 16)] = accs[j]
        pltpu.sync_copy(den2v, den2parts.at[c, s, t])

        @pl.when(s == 0)
        def _():
            pltpu.sync_copy(out2acc, out2parts.at[c, t])
        return _
    lax.fori_loop(0, T, t_body, None)


def _sc2_layer2(h1flat, s2s, s2d, m2, srcp, dstp, bidx):
    mesh = plsc.VectorSubcoreMesh(core_axis_name="c", subcore_axis_name="s")
    f = pl.kernel(
        _sc2_body,
        out_type=(
            jax.ShapeDtypeStruct((NC, T, 16, H), jnp.float32),
            jax.ShapeDtypeStruct((NC, NS, T, 8, 16), jnp.float32),
        ),
        mesh=mesh,
        compiler_params=pltpu.CompilerParams(needs_layout_passes=False,
                                             use_tc_tiling_on_sc=False),
        scratch_types=[
            pltpu.VMEM((ECA,), jnp.int32),
            pltpu.VMEM((ECA,), jnp.int32),
            pltpu.VMEM((N,), jnp.float32),
            pltpu.VMEM((16,), jnp.float32),
            pltpu.VMEM((16,), jnp.float32),
            pltpu.VMEM((16,), jnp.int32),
            pltpu.VMEM((8, 16), jnp.float32),
            pltpu.VMEM((16,), jnp.int32),
            pltpu.VMEM((16,), jnp.int32),
            pltpu.VMEM((16, H), jnp.float32),
            pltpu.VMEM((16, H), jnp.float32),
            pltpu.VMEM((16, H), jnp.float32),
            pltpu.SemaphoreType.DMA,
            pltpu.VMEM_SHARED((16, H), jnp.float32),
        ],
    )
    return f(h1flat, s2s, s2d, m2, srcp, dstp, bidx)


NB = 1000


def _proj_body(era5_ref, bc_ref, bd_ref, wera_ref, wbc_ref, wbd_ref, bias_ref,
               v1s_ref, v1d_ref, xA_ref, xB_ref, s1s_ref, s1d_ref):
    x = (jnp.dot(era5_ref[0], wera_ref[...], preferred_element_type=jnp.float32)
         + jnp.dot(bc_ref[...], wbc_ref[...], preferred_element_type=jnp.float32)
         + jnp.dot(bd_ref[...], wbd_ref[...], preferred_element_type=jnp.float32)
         + bias_ref[...])
    x = jax.nn.relu(x)
    z16c = jnp.zeros((NB, 16), jnp.float32)
    xA_ref[0] = jnp.concatenate([x[:, :64], z16c], axis=1)
    xB_ref[0] = jnp.concatenate([x[:, 64:], jnp.ones((NB, 1), jnp.float32),
                                 z16c[:, :15]], axis=1)
    s1s_ref[0] = jnp.dot(x, v1s_ref[...], preferred_element_type=jnp.float32)
    s1d_ref[0] = jnp.dot(x, v1d_ref[...], preferred_element_type=jnp.float32)


def _proj(era5t, bc, bd, wera, wbc, wbd, bias, v1s, v1d):
    return pl.pallas_call(
        _proj_body,
        grid=(T, N // NB),
        in_specs=[
            pl.BlockSpec((1, NB, D_ERA), lambda t, nb: (t, nb, 0)),
            pl.BlockSpec((NB, D_BC), lambda t, nb: (nb, 0)),
            pl.BlockSpec((NB, D_BD), lambda t, nb: (nb, 0)),
            pl.BlockSpec((D_ERA, H), lambda t, nb: (0, 0)),
            pl.BlockSpec((D_BC, H), lambda t, nb: (0, 0)),
            pl.BlockSpec((D_BD, H), lambda t, nb: (0, 0)),
            pl.BlockSpec((1, H), lambda t, nb: (0, 0)),
            pl.BlockSpec((H, 1), lambda t, nb: (0, 0)),
            pl.BlockSpec((H, 1), lambda t, nb: (0, 0)),
        ],
        out_specs=[
            pl.BlockSpec((1, NB, HC), lambda t, nb: (t, nb, 0)),
            pl.BlockSpec((1, NB, HC), lambda t, nb: (t, nb, 0)),
            pl.BlockSpec((1, NB, 1), lambda t, nb: (t, nb, 0)),
            pl.BlockSpec((1, NB, 1), lambda t, nb: (t, nb, 0)),
        ],
        out_shape=[
            jax.ShapeDtypeStruct((T, N, HC), jnp.float32),
            jax.ShapeDtypeStruct((T, N, HC), jnp.float32),
            jax.ShapeDtypeStruct((T, N, 1), jnp.float32),
            jax.ShapeDtypeStruct((T, N, 1), jnp.float32),
        ],
    )(era5t, bc, bd, wera, wbc, wbd, bias, v1s, v1d)


def _h1_body(pa_ref, pb_ref, w_ref, b_ref, v2s_ref, h1_ref, s2s_ref):
    pa = pa_ref[0]
    pb = pb_ref[0]
    den = pb[:, 64:65] + 1e-16
    P1 = jnp.concatenate([pa[:, :64], pb[:, :64]], axis=1) / den
    h1 = jnp.dot(P1, w_ref[...], preferred_element_type=jnp.float32) + b_ref[...]
    h1 = jnp.where(h1 > 0, h1, jnp.exp(jnp.minimum(h1, 0.0)) - 1.0)
    h1_ref[0] = h1
    s2s_ref[0] = jnp.dot(h1, v2s_ref[...], preferred_element_type=jnp.float32)


def _h1k(pA, pB, w, b, v2s):
    return pl.pallas_call(
        _h1_body,
        grid=(T, N // NB),
        in_specs=[
            pl.BlockSpec((1, NB, HC), lambda t, nb: (t, nb, 0)),
            pl.BlockSpec((1, NB, HC), lambda t, nb: (t, nb, 0)),
            pl.BlockSpec((H, H), lambda t, nb: (0, 0)),
            pl.BlockSpec((1, H), lambda t, nb: (0, 0)),
            pl.BlockSpec((H, 1), lambda t, nb: (0, 0)),
        ],
        out_specs=[
            pl.BlockSpec((1, NB, H), lambda t, nb: (t, nb, 0)),
            pl.BlockSpec((1, NB, 1), lambda t, nb: (t, nb, 0)),
        ],
        out_shape=[
            jax.ShapeDtypeStruct((T, N, H), jnp.float32),
            jax.ShapeDtypeStruct((T, N, 1), jnp.float32),
        ],
    )(pA, pB, w, b, v2s)


def _lstm_head_body(series_ref, wih_ref, whh_ref, b_ref, hw_ref, hb_ref,
                    cast_ref, h_ref, c_ref):
    h = jnp.zeros((B, LH), jnp.float32)
    c = jnp.zeros((B, LH), jnp.float32)
    wih = wih_ref[...]
    whh = whh_ref[...]
    b = b_ref[...]
    hw = hw_ref[...]
    hb = hb_ref[...]
    for t in range(T):
        x_t = series_ref[t]
        z = jnp.dot(x_t, wih, preferred_element_type=jnp.float32) + \
            jnp.dot(h, whh, preferred_element_type=jnp.float32) + b
        i = jax.nn.sigmoid(z[:, 0 * LH:1 * LH])
        f = jax.nn.sigmoid(z[:, 1 * LH:2 * LH])
        g = jnp.tanh(z[:, 2 * LH:3 * LH])
        o = jax.nn.sigmoid(z[:, 3 * LH:4 * LH])
        c = f * c + i * g
        h = o * jnp.tanh(c)
        zc = jnp.dot(h, hw, preferred_element_type=jnp.float32) + hb
        m_ = zc[:, 0:K]
        b_ = jax.nn.softplus(zc[:, K:2 * K]) + 1e-5
        t_ = jax.nn.sigmoid(zc[:, 2 * K:3 * K])
        p_ = jax.nn.softmax(zc[:, 3 * K:4 * K], axis=-1)
        cast_ref[t] = jnp.concatenate([m_, b_, t_, p_], axis=-1)
    h_ref[...] = h
    c_ref[...] = c


def _lstm_head(series_tbh, W_ih, W_hh, b_lstm, head_W, head_b):
    return pl.pallas_call(
        _lstm_head_body,
        out_shape=(
            jax.ShapeDtypeStruct((T, B, 4 * K), jnp.float32),
            jax.ShapeDtypeStruct((B, LH), jnp.float32),
            jax.ShapeDtypeStruct((B, LH), jnp.float32),
        ),
    )(series_tbh, W_ih, W_hh, b_lstm, head_W, head_b)


def kernel(era5, basinContinuous, basinDiscrete, riverContinuous, riverDiscrete,
           bp_Wc, bp_bc, bp_Wd, bp_bd,
           g1_W, g1_as, g1_ad, g1_b, g2_W, g2_as, g2_ad, g2_b,
           rp_Wc, rp_bc, rp_Wd, rp_bd,
           W_ih, W_hh, b_lstm, head_W, head_b,
           edge_index, nodes):
    src, dst = edge_index[0], edge_index[1]

    era5t = jnp.swapaxes(era5, 0, 1)
    W_era = bp_Wc[:D_ERA]
    W_bc = bp_Wc[D_ERA:]
    bias = (bp_bc + bp_bd)[None]
    v1s = (g1_W @ g1_as)[:, None]
    v1d = (g1_W @ g1_ad)[:, None]
    xA3, xB3, s1s3, s1d3 = _proj(era5t, basinContinuous, basinDiscrete,
                                 W_era, W_bc, bp_Wd, bias, v1s, v1d)
    s1s = s1s3[..., 0]
    s1d = s1d3[..., 0]
    M1 = jnp.max(s1s, axis=1) + jnp.max(s1d, axis=1)
    m1bc = jnp.broadcast_to(M1[:, None], (T, 16))

    srcp = jnp.pad(src.reshape(NS, EC), ((0, 0), (0, ECA - EC)))
    dstp = jnp.pad(dst.reshape(NS, EC), ((0, 0), (0, ECA - EC)))
    p1parts = _sc1_layer1(xA3.reshape(T * N, HC), xB3.reshape(T * N, HC),
                          s1s, s1d, m1bc, srcp, dstp)

    v2s = (g2_W @ g2_as)[:, None]
    h1, s2s3 = _h1k(p1parts[0], p1parts[1], g1_W, g1_b[None], v2s)
    s2s = s2s3[..., 0]

    batchIndices = jnp.concatenate([jnp.zeros((1,), nodes.dtype), jnp.cumsum(nodes)[:-1]])

    v2d = g2_W @ g2_ad
    s2d_sel = h1[:, batchIndices, :] @ v2d
    M2 = jnp.max(s2s, axis=1) + jnp.max(s2d_sel, axis=1)

    s2d_pad = jnp.pad(s2d_sel, ((0, 0), (0, 8)))
    m2bc = jnp.broadcast_to(M2[:, None], (T, 16))
    bidx_pad = jnp.pad(batchIndices.astype(jnp.int32), (0, 8),
                       constant_values=-1)
    out2parts, den2parts = _sc2_layer2(h1.reshape(T * N, H), s2s, s2d_pad,
                                       m2bc, srcp, dstp, bidx_pad)
    agg = out2parts.sum(axis=0)[:, :8, :]
    denom2 = den2parts.sum(axis=(0, 1, 4))
    out2 = (agg / (denom2[..., None] + 1e-16)) @ g2_W + g2_b
    first = jnp.argmax(batchIndices[None, :] == batchIndices[:, None], axis=1)
    out2 = out2[:, first, :]

    rcat = jnp.concatenate([out2, jnp.broadcast_to(riverContinuous[None], (T, B, D_RC))], -1)
    series = jax.nn.relu(rcat @ rp_Wc + rp_bc + riverDiscrete @ rp_Wd + rp_bd)

    cast_t, h, c = _lstm_head(series, W_ih, W_hh, b_lstm, head_W, head_b)
    cast = jnp.swapaxes(cast_t, 0, 1)
    return cast, (h, c)


# speedup vs baseline: 1.0602x; 1.0118x over previous
import jax
import jax.numpy as jnp
from jax import lax
from jax.experimental import pallas as pl
from jax.experimental.pallas import tpu as pltpu
from jax.experimental.pallas import tpu_sc as plsc

N = 10000; T = 16; E = 160000; B = 8
D_ERA = 16; D_BC = 32; D_BD = 16; D_RC = 16; D_RD = 8
H = 128; LH = 256; K = 3

NC = 2
NS = 16
EC = E // NS
ECP = 10112
NG = ECP // 16
ECA = ECP + 128
HC = 80
NP = 10240
NPT = NP // NS


def _sc1_body(xA, xB, s1s, s1d, m1, srcp, dstp,
              p1parts,
              srcc, dstc, s1sv, s1dv, m1v,
              idxS, idxD, rows, scaled,
              zb, gsems, ssems,
              p1acc):
    c = lax.axis_index("c")
    s = lax.axis_index("s")
    z16 = jnp.zeros((16,), jnp.float32)
    iota = lax.iota(jnp.int32, 16)
    D = 8

    pltpu.sync_copy(srcp.at[s], srcc)
    pltpu.sync_copy(dstp.at[s], dstc)

    def zero_zb(i, _):
        for k in range(HC // 16):
            zb[i, pl.ds(k * 16, 16)] = z16
        return _
    lax.fori_loop(0, 80, zero_zb, None)

    def gissue(g, toff, b):
        idxS[b, pl.ds(0, 16)] = srcc[pl.ds(g * 16, 16)] + toff
        @pl.when(c == 0)
        def _():
            pltpu.async_copy(xA.at[idxS.at[b]], rows.at[b], gsems.at[b])
        @pl.when(c == 1)
        def _():
            pltpu.async_copy(xB.at[idxS.at[b]], rows.at[b], gsems.at[b])

    def gwait(b):
        @pl.when(c == 0)
        def _():
            pltpu.make_async_copy(xA.at[idxS.at[b]], rows.at[b],
                                  gsems.at[b]).wait()
        @pl.when(c == 1)
        def _():
            pltpu.make_async_copy(xB.at[idxS.at[b]], rows.at[b],
                                  gsems.at[b]).wait()

    def swait(b):
        pltpu.make_async_copy(scaled.at[b], p1acc.at[idxD.at[b]],
                              ssems.at[b]).wait()

    def t_body(t, _):
        pltpu.sync_copy(s1s.at[t], s1sv)
        pltpu.sync_copy(s1d.at[t], s1dv)
        pltpu.sync_copy(m1.at[t], m1v)
        m1t = m1v[...]
        toff = t * N

        for j in range(8):
            dst_off = pl.multiple_of(s * NPT + j * 80, 8)
            pltpu.sync_copy(zb, p1acc.at[pl.ds(dst_off, 80), :])
        plsc.subcore_barrier()

        for b in range(D):
            gissue(b, toff, b)

        def quad_body(q, _):
            for b in range(D):
                g = q * D + b
                gwait(b)
                @pl.when(q > 0)
                def _():
                    swait(b)
                off = g * 16
                src16 = srcc[pl.ds(off, 16)]
                dst16 = dstc[pl.ds(off, 16)]
                s_s = plsc.load_gather(s1sv, [src16])
                s_d = plsc.load_gather(s1dv, [dst16])
                e = s_s + s_d
                e = jnp.where(e >= 0.0, e, e * 0.2)
                ex = jnp.exp(e - m1t)
                ex = jnp.where((off + iota) < EC, ex, 0.0)
                for j in range(16):
                    exj = ex[j]
                    for k in range(HC // 16):
                        sl = pl.ds(k * 16, 16)
                        scaled[b, j, sl] = rows[b, j, sl] * exj
                idxD[b, pl.ds(0, 16)] = dst16
                pltpu.async_copy(scaled.at[b], p1acc.at[idxD.at[b]],
                                 ssems.at[b], add=True)
                gissue(g + D, toff, b)
            return _
        lax.fori_loop(0, NG // D, quad_body, None)
        for b in range(D):
            gwait(b)
            swait(b)
        plsc.subcore_barrier()

        src_off = pl.multiple_of(s * NPT, 8)
        pltpu.sync_copy(p1acc.at[pl.ds(src_off, NPT), :],
                        p1parts.at[c, t, pl.ds(src_off, NPT), :])
        return _
    lax.fori_loop(0, T, t_body, None)


def _sc1_layer1(xA, xB, s1s, s1d, m1, srcp, dstp):
    mesh = plsc.VectorSubcoreMesh(core_axis_name="c", subcore_axis_name="s")
    f = pl.kernel(
        _sc1_body,
        out_type=jax.ShapeDtypeStruct((NC, T, NP, HC), jnp.float32),
        mesh=mesh,
        compiler_params=pltpu.CompilerParams(needs_layout_passes=False,
                                             use_tc_tiling_on_sc=False),
        scratch_types=[
            pltpu.VMEM((ECA,), jnp.int32),
            pltpu.VMEM((ECA,), jnp.int32),
            pltpu.VMEM((N,), jnp.float32),
            pltpu.VMEM((N,), jnp.float32),
            pltpu.VMEM((16,), jnp.float32),
            pltpu.VMEM((8, 16), jnp.int32),
            pltpu.VMEM((8, 16), jnp.int32),
            pltpu.VMEM((8, 16, HC), jnp.float32),
            pltpu.VMEM((8, 16, HC), jnp.float32),
            pltpu.VMEM((80, HC), jnp.float32),
            pltpu.SemaphoreType.DMA((8,)),
            pltpu.SemaphoreType.DMA((8,)),
            pltpu.VMEM_SHARED((NP, HC), jnp.float32),
        ],
    )
    return f(xA, xB, s1s, s1d, m1, srcp, dstp)


NGC = 313


def _sc2_body(h1flat, s2s, s2d, m2, srcp, dstp, bidx,
              out2parts, den2parts,
              srcc, dstc, s2sv, s2dv, m2v, bidxv, den2v,
              idxS, slotb, rows, scaled, zb2, sem,
              out2acc):
    c = lax.axis_index("c")
    s = lax.axis_index("s")
    z16 = jnp.zeros((16,), jnp.float32)
    iota = lax.iota(jnp.int32, 16)

    pltpu.sync_copy(srcp.at[s], srcc)
    pltpu.sync_copy(dstp.at[s], dstc)
    pltpu.sync_copy(bidx, bidxv)
    for k in range(16):
        for q in range(8):
            zb2[k, pl.ds(q * 16, 16)] = z16
    bvec = bidxv[...]
    base = c * (EC // 2)
    limit = base + (EC // 2)

    def t_body(t, _):
        pltpu.sync_copy(s2s.at[t], s2sv)
        pltpu.sync_copy(s2d.at[t], s2dv)
        pltpu.sync_copy(m2.at[t], m2v)
        m2t = m2v[...]
        toff = t * N

        @pl.when(s == 0)
        def _():
            pltpu.sync_copy(zb2, out2acc)
        plsc.subcore_barrier()

        def g_body(g, accs):
            off = base + g * 16
            dst16 = dstc[pl.ds(off, 16)]
            valid = (off + iota) < limit
            hit = valid
            m = jnp.zeros((16,), jnp.bool_)
            for j in range(8):
                m = m | (dst16 == bvec[j])
            hit = hit & m
            anyhit = jnp.max(jnp.where(hit, 1, 0), axis=0)

            def do_group(accs):
                src16 = srcc[pl.ds(off, 16)]
                slot = jnp.full((16,), 8, jnp.int32)
                for j in range(7, -1, -1):
                    slot = jnp.where(dst16 == bvec[j], j, slot)
                s_s = plsc.load_gather(s2sv, [src16])
                s_d = plsc.load_gather(s2dv, [slot])
                e = s_s + s_d
                e = jnp.where(e >= 0.0, e, e * 0.2)
                ex = jnp.exp(e - m2t)
                ex = jnp.where(hit, ex, 0.0)
                naccs = tuple(accs[j] + jnp.where(slot == j, ex, 0.0)
                              for j in range(8))
                idxS[...] = src16 + toff
                slotb[...] = slot
                pltpu.async_copy(h1flat.at[idxS], rows, sem).wait()
                for j in range(16):
                    exj = ex[j]
                    for k in range(8):
                        sl = pl.ds(k * 16, 16)
                        scaled[j, sl] = rows[j, sl] * exj
                pltpu.sync_copy(scaled, out2acc.at[slotb], add=True)
                return naccs

            def skip(accs):
                return accs
            return lax.cond(anyhit > 0, do_group, skip, accs)

        accs = tuple(z16 for _ in range(8))
        accs = lax.fori_loop(0, NGC, g_body, accs)
        plsc.subcore_barrier()

        for j in range(8):
            den2v[j, pl.ds(0, 16)] = accs[j]
        pltpu.sync_copy(den2v, den2parts.at[c, s, t])

        @pl.when(s == 0)
        def _():
            pltpu.sync_copy(out2acc, out2parts.at[c, t])
        return _
    lax.fori_loop(0, T, t_body, None)


def _sc2_layer2(h1flat, s2s, s2d, m2, srcp, dstp, bidx):
    mesh = plsc.VectorSubcoreMesh(core_axis_name="c", subcore_axis_name="s")
    f = pl.kernel(
        _sc2_body,
        out_type=(
            jax.ShapeDtypeStruct((NC, T, 16, H), jnp.float32),
            jax.ShapeDtypeStruct((NC, NS, T, 8, 16), jnp.float32),
        ),
        mesh=mesh,
        compiler_params=pltpu.CompilerParams(needs_layout_passes=False,
                                             use_tc_tiling_on_sc=False),
        scratch_types=[
            pltpu.VMEM((ECA,), jnp.int32),
            pltpu.VMEM((ECA,), jnp.int32),
            pltpu.VMEM((N,), jnp.float32),
            pltpu.VMEM((16,), jnp.float32),
            pltpu.VMEM((16,), jnp.float32),
            pltpu.VMEM((16,), jnp.int32),
            pltpu.VMEM((8, 16), jnp.float32),
            pltpu.VMEM((16,), jnp.int32),
            pltpu.VMEM((16,), jnp.int32),
            pltpu.VMEM((16, H), jnp.float32),
            pltpu.VMEM((16, H), jnp.float32),
            pltpu.VMEM((16, H), jnp.float32),
            pltpu.SemaphoreType.DMA,
            pltpu.VMEM_SHARED((16, H), jnp.float32),
        ],
    )
    return f(h1flat, s2s, s2d, m2, srcp, dstp, bidx)


NB = 1000


def _proj_body(era5_ref, bc_ref, bd_ref, wera_ref, wbc_ref, wbd_ref, bias_ref,
               v1s_ref, v1d_ref, xA_ref, xB_ref, s1s_ref, s1d_ref):
    x = (jnp.dot(era5_ref[0], wera_ref[...], preferred_element_type=jnp.float32)
         + jnp.dot(bc_ref[...], wbc_ref[...], preferred_element_type=jnp.float32)
         + jnp.dot(bd_ref[...], wbd_ref[...], preferred_element_type=jnp.float32)
         + bias_ref[...])
    x = jax.nn.relu(x)
    z16c = jnp.zeros((NB, 16), jnp.float32)
    xA_ref[0] = jnp.concatenate([x[:, :64], z16c], axis=1)
    xB_ref[0] = jnp.concatenate([x[:, 64:], jnp.ones((NB, 1), jnp.float32),
                                 z16c[:, :15]], axis=1)
    s1s_ref[0] = jnp.dot(x, v1s_ref[...], preferred_element_type=jnp.float32)
    s1d_ref[0] = jnp.dot(x, v1d_ref[...], preferred_element_type=jnp.float32)


def _proj(era5t, bc, bd, wera, wbc, wbd, bias, v1s, v1d):
    return pl.pallas_call(
        _proj_body,
        grid=(T, N // NB),
        in_specs=[
            pl.BlockSpec((1, NB, D_ERA), lambda t, nb: (t, nb, 0)),
            pl.BlockSpec((NB, D_BC), lambda t, nb: (nb, 0)),
            pl.BlockSpec((NB, D_BD), lambda t, nb: (nb, 0)),
            pl.BlockSpec((D_ERA, H), lambda t, nb: (0, 0)),
            pl.BlockSpec((D_BC, H), lambda t, nb: (0, 0)),
            pl.BlockSpec((D_BD, H), lambda t, nb: (0, 0)),
            pl.BlockSpec((1, H), lambda t, nb: (0, 0)),
            pl.BlockSpec((H, 1), lambda t, nb: (0, 0)),
            pl.BlockSpec((H, 1), lambda t, nb: (0, 0)),
        ],
        out_specs=[
            pl.BlockSpec((1, NB, HC), lambda t, nb: (t, nb, 0)),
            pl.BlockSpec((1, NB, HC), lambda t, nb: (t, nb, 0)),
            pl.BlockSpec((1, NB, 1), lambda t, nb: (t, nb, 0)),
            pl.BlockSpec((1, NB, 1), lambda t, nb: (t, nb, 0)),
        ],
        out_shape=[
            jax.ShapeDtypeStruct((T, N, HC), jnp.float32),
            jax.ShapeDtypeStruct((T, N, HC), jnp.float32),
            jax.ShapeDtypeStruct((T, N, 1), jnp.float32),
            jax.ShapeDtypeStruct((T, N, 1), jnp.float32),
        ],
    )(era5t, bc, bd, wera, wbc, wbd, bias, v1s, v1d)


def _h1_body(pa_ref, pb_ref, w_ref, b_ref, v2s_ref, h1_ref, s2s_ref):
    pa = pa_ref[0]
    pb = pb_ref[0]
    den = pb[:, 64:65] + 1e-16
    P1 = jnp.concatenate([pa[:, :64], pb[:, :64]], axis=1) / den
    h1 = jnp.dot(P1, w_ref[...], preferred_element_type=jnp.float32) + b_ref[...]
    h1 = jnp.where(h1 > 0, h1, jnp.exp(jnp.minimum(h1, 0.0)) - 1.0)
    h1_ref[0] = h1
    s2s_ref[0] = jnp.dot(h1, v2s_ref[...], preferred_element_type=jnp.float32)


def _h1k(pA, pB, w, b, v2s):
    return pl.pallas_call(
        _h1_body,
        grid=(T, N // NB),
        in_specs=[
            pl.BlockSpec((1, NB, HC), lambda t, nb: (t, nb, 0)),
            pl.BlockSpec((1, NB, HC), lambda t, nb: (t, nb, 0)),
            pl.BlockSpec((H, H), lambda t, nb: (0, 0)),
            pl.BlockSpec((1, H), lambda t, nb: (0, 0)),
            pl.BlockSpec((H, 1), lambda t, nb: (0, 0)),
        ],
        out_specs=[
            pl.BlockSpec((1, NB, H), lambda t, nb: (t, nb, 0)),
            pl.BlockSpec((1, NB, 1), lambda t, nb: (t, nb, 0)),
        ],
        out_shape=[
            jax.ShapeDtypeStruct((T, N, H), jnp.float32),
            jax.ShapeDtypeStruct((T, N, 1), jnp.float32),
        ],
    )(pA, pB, w, b, v2s)


def _lstm_head_body(series_ref, wih_ref, whh_ref, b_ref, hw_ref, hb_ref,
                    cast_ref, h_ref, c_ref):
    h = jnp.zeros((B, LH), jnp.float32)
    c = jnp.zeros((B, LH), jnp.float32)
    wih = wih_ref[...]
    whh = whh_ref[...]
    b = b_ref[...]
    hw = hw_ref[...]
    hb = hb_ref[...]
    for t in range(T):
        x_t = series_ref[t]
        z = jnp.dot(x_t, wih, preferred_element_type=jnp.float32) + \
            jnp.dot(h, whh, preferred_element_type=jnp.float32) + b
        i = jax.nn.sigmoid(z[:, 0 * LH:1 * LH])
        f = jax.nn.sigmoid(z[:, 1 * LH:2 * LH])
        g = jnp.tanh(z[:, 2 * LH:3 * LH])
        o = jax.nn.sigmoid(z[:, 3 * LH:4 * LH])
        c = f * c + i * g
        h = o * jnp.tanh(c)
        zc = jnp.dot(h, hw, preferred_element_type=jnp.float32) + hb
        m_ = zc[:, 0:K]
        b_ = jax.nn.softplus(zc[:, K:2 * K]) + 1e-5
        t_ = jax.nn.sigmoid(zc[:, 2 * K:3 * K])
        p_ = jax.nn.softmax(zc[:, 3 * K:4 * K], axis=-1)
        cast_ref[t] = jnp.concatenate([m_, b_, t_, p_], axis=-1)
    h_ref[...] = h
    c_ref[...] = c


def _lstm_head(series_tbh, W_ih, W_hh, b_lstm, head_W, head_b):
    return pl.pallas_call(
        _lstm_head_body,
        out_shape=(
            jax.ShapeDtypeStruct((T, B, 4 * K), jnp.float32),
            jax.ShapeDtypeStruct((B, LH), jnp.float32),
            jax.ShapeDtypeStruct((B, LH), jnp.float32),
        ),
    )(series_tbh, W_ih, W_hh, b_lstm, head_W, head_b)


def kernel(era5, basinContinuous, basinDiscrete, riverContinuous, riverDiscrete,
           bp_Wc, bp_bc, bp_Wd, bp_bd,
           g1_W, g1_as, g1_ad, g1_b, g2_W, g2_as, g2_ad, g2_b,
           rp_Wc, rp_bc, rp_Wd, rp_bd,
           W_ih, W_hh, b_lstm, head_W, head_b,
           edge_index, nodes):
    src, dst = edge_index[0], edge_index[1]

    era5t = jnp.swapaxes(era5, 0, 1)
    W_era = bp_Wc[:D_ERA]
    W_bc = bp_Wc[D_ERA:]
    bias = (bp_bc + bp_bd)[None]
    v1s = (g1_W @ g1_as)[:, None]
    v1d = (g1_W @ g1_ad)[:, None]
    xA3, xB3, s1s3, s1d3 = _proj(era5t, basinContinuous, basinDiscrete,
                                 W_era, W_bc, bp_Wd, bias, v1s, v1d)
    s1s = s1s3[..., 0]
    s1d = s1d3[..., 0]
    M1 = jnp.max(s1s, axis=1) + jnp.max(s1d, axis=1)
    m1bc = jnp.broadcast_to(M1[:, None], (T, 16))

    srcp = jnp.pad(src.reshape(NS, EC), ((0, 0), (0, ECA - EC)))
    dstp = jnp.pad(dst.reshape(NS, EC), ((0, 0), (0, ECA - EC)))
    p1parts = _sc1_layer1(xA3.reshape(T * N, HC), xB3.reshape(T * N, HC),
                          s1s, s1d, m1bc, srcp, dstp)

    v2s = (g2_W @ g2_as)[:, None]
    h1, s2s3 = _h1k(p1parts[0], p1parts[1], g1_W, g1_b[None], v2s)
    s2s = s2s3[..., 0]

    batchIndices = jnp.concatenate([jnp.zeros((1,), nodes.dtype), jnp.cumsum(nodes)[:-1]])

    v2d = g2_W @ g2_ad
    s2d_sel = h1[:, batchIndices, :] @ v2d
    M2 = jnp.max(s2s, axis=1) + jnp.max(s2d_sel, axis=1)

    s2d_pad = jnp.pad(s2d_sel, ((0, 0), (0, 8)))
    m2bc = jnp.broadcast_to(M2[:, None], (T, 16))
    bidx_pad = jnp.pad(batchIndices.astype(jnp.int32), (0, 8),
                       constant_values=-1)
    out2parts, den2parts = _sc2_layer2(h1.reshape(T * N, H), s2s, s2d_pad,
                                       m2bc, srcp, dstp, bidx_pad)
    agg = out2parts.sum(axis=0)[:, :8, :]
    denom2 = den2parts.sum(axis=(0, 1, 4))
    out2 = (agg / (denom2[..., None] + 1e-16)) @ g2_W + g2_b
    first = jnp.argmax(batchIndices[None, :] == batchIndices[:, None], axis=1)
    out2 = out2[:, first, :]

    rcat = jnp.concatenate([out2, jnp.broadcast_to(riverContinuous[None], (T, B, D_RC))], -1)
    series = jax.nn.relu(rcat @ rp_Wc + rp_bc + riverDiscrete @ rp_Wd + rp_bd)

    cast_t, h, c = _lstm_head(series, W_ih, W_hh, b_lstm, head_W, head_b)
    cast = jnp.swapaxes(cast_t, 0, 1)
    return cast, (h, c)
